# baseline (speedup 1.0000x reference)
import jax
import jax.numpy as jnp
from jax import lax
from jax.experimental import pallas as pl
from jax.experimental.pallas import tpu as pltpu
from jax.experimental.pallas import tpu_sc as plsc

NUM_WORKERS = 32
BPW = 128
NCB = 8
D_PAD = NCB * 128


def _make_sc_gather(b, t, d):
    d_tail = d - (NCB - 1) * 128
    mesh = plsc.VectorSubcoreMesh(core_axis_name="c", subcore_axis_name="s")

    @pl.kernel(
        mesh=mesh,
        compiler_params=pltpu.CompilerParams(needs_layout_passes=False),
        out_type=jax.ShapeDtypeStruct((t, d, b), jnp.float32),
        scratch_types=[
            pltpu.VMEM((t, NCB, BPW), jnp.int32),
            pltpu.VMEM((BPW, 128), jnp.float32),
            pltpu.VMEM((BPW, 128), jnp.float32),
            pltpu.VMEM((128, BPW), jnp.float32),
            pltpu.VMEM((128, BPW), jnp.float32),
            pltpu.SemaphoreType.DMA,
            pltpu.SemaphoreType.DMA,
            pltpu.SemaphoreType.DMA,
            pltpu.SemaphoreType.DMA,
        ],
    )
    def sc_gather(t8_hbm, idx_hbm, out_hbm, idx_v, pga, pgb, pta, ptb,
                  sem_ga, sem_gb, sem_wa, sem_wb):
        wid = lax.axis_index("s") * 2 + lax.axis_index("c")
        bb = wid * BPW
        pltpu.sync_copy(idx_hbm.at[wid], idx_v)

        def gather(tt, c, buf, sem):
            return pltpu.make_async_copy(t8_hbm.at[idx_v.at[tt, c]], buf, sem)

        def cw(c):
            return d_tail if c == NCB - 1 else 128

        def write(tt, c, buf, sem):
            return pltpu.make_async_copy(
                buf.at[pl.ds(0, cw(c))],
                out_hbm.at[tt, pl.ds(c * 128, cw(c)), pl.ds(bb, BPW)], sem)

        rows = [lax.iota(jnp.int32, 16) + 16 * j for j in range(BPW // 16)]

        def transpose(pg, pt, width):
            def col(dd, carry):
                cols = jnp.full((16,), dd, jnp.int32)
                for j in range(BPW // 16):
                    pt[dd, pl.ds(16 * j, 16)] = plsc.load_gather(
                        pg, [rows[j], cols])
                return carry
            lax.fori_loop(0, width, col, 0)

        gather(0, 0, pga, sem_ga).start()

        def body(tt, carry):
            for c in range(NCB):
                pg, sg = (pga, sem_ga) if c % 2 == 0 else (pgb, sem_gb)
                pt, sw = (pta, sem_wa) if c % 2 == 0 else (ptb, sem_wb)
                gather(tt, c, pg, sg).wait()
                if c < NCB - 1:
                    pgn, sgn = (pgb, sem_gb) if c % 2 == 0 else (pga, sem_ga)
                    gather(tt, c + 1, pgn, sgn).start()
                else:
                    @pl.when(tt < t - 1)
                    def _():
                        gather(tt + 1, 0, pga, sem_ga).start()
                if c >= 2:
                    write(tt, c - 2, pt, sw).wait()
                else:
                    @pl.when(tt > 0)
                    def _():
                        write(tt - 1, NCB - 2 + c, pt, sw).wait()
                transpose(pg, pt, cw(c))
                write(tt, c, pt, sw).start()
            return carry

        lax.fori_loop(0, t, body, 0)
        write(t - 1, NCB - 2, pta, sem_wa).wait()
        write(t - 1, NCB - 1, ptb, sem_wb).wait()

    return sc_gather


def kernel(x, embedding):
    b, t = x.shape
    v, d = embedding.shape
    t8 = jnp.pad(embedding, ((0, 0), (0, D_PAD - d))).reshape(v * NCB, 128)
    xr = x.astype(jnp.int32).reshape(NUM_WORKERS, BPW, t).transpose(0, 2, 1)
    gidx = (xr[:, :, None, :] * NCB
            + jnp.arange(NCB, dtype=jnp.int32)[None, None, :, None])
    out_t = _make_sc_gather(b, t, d)(t8, gidx)
    return jnp.transpose(out_t, (2, 0, 1))


# speedup vs baseline: 2.9417x; 2.9417x over previous
import jax
import jax.numpy as jnp
from jax import lax
from jax.experimental import pallas as pl
from jax.experimental.pallas import tpu as pltpu
from jax.experimental.pallas import tpu_sc as plsc

NUM_WORKERS = 32
BPW = 128
NCB = 8
D_PAD = NCB * 128


def _make_sc_gather(b, t, d):
    d_tail = d - (NCB - 1) * 128
    mesh = plsc.VectorSubcoreMesh(core_axis_name="c", subcore_axis_name="s")

    @pl.kernel(
        mesh=mesh,
        out_type=jax.ShapeDtypeStruct((b, t, d), jnp.float32),
        scratch_types=[
            pltpu.VMEM((t, NCB, BPW), jnp.int32),
            pltpu.VMEM((BPW, 128), jnp.float32),
            pltpu.VMEM((BPW, 128), jnp.float32),
            pltpu.VMEM((BPW, d_tail), jnp.float32),
            pltpu.VMEM((BPW, d_tail), jnp.float32),
            pltpu.SemaphoreType.DMA,
            pltpu.SemaphoreType.DMA,
            pltpu.SemaphoreType.DMA,
            pltpu.SemaphoreType.DMA,
            pltpu.SemaphoreType.DMA,
            pltpu.SemaphoreType.DMA,
        ],
    )
    def sc_gather(t8_hbm, idx_hbm, out_hbm, idx_v, pa, pb, buf7a, buf7b,
                  sem_ga, sem_gb, sem_wa, sem_wb, sem_7a, sem_7b):
        wid = lax.axis_index("s") * 2 + lax.axis_index("c")
        bb = wid * BPW
        pltpu.sync_copy(idx_hbm.at[wid], idx_v)

        def gather(tt, c, buf, sem):
            return pltpu.make_async_copy(t8_hbm.at[idx_v.at[tt, c]], buf, sem)

        def write(tt, c, buf, sem):
            return pltpu.make_async_copy(
                buf, out_hbm.at[pl.ds(bb, BPW), tt, pl.ds(c * 128, 128)], sem)

        def write7(tt, bf, sem):
            return pltpu.make_async_copy(
                bf, out_hbm.at[pl.ds(bb, BPW), tt, pl.ds((NCB - 1) * 128,
                                                         d_tail)], sem)

        def tail(tt, bf, sem):
            @pl.when(tt >= 2)
            def _():
                write7(tt - 2, bf, sem).wait()

            def row_copy(r, carry):
                for kk in range(d_tail // 16):
                    bf[r, pl.ds(kk * 16, 16)] = pb[r, pl.ds(kk * 16, 16)]
                bf[r, pl.ds(d_tail - 16, 16)] = pb[r, pl.ds(d_tail - 16, 16)]
                return carry
            lax.fori_loop(0, BPW, row_copy, 0)
            write7(tt, bf, sem).start()

        gather(0, 0, pa, sem_ga).start()

        def body(tt, carry):
            for p in range(4):
                ca, cb = 2 * p, 2 * p + 1
                gather(tt, ca, pa, sem_ga).wait()
                write(tt, ca, pa, sem_wa).start()
                if p > 0:
                    write(tt, cb - 2, pb, sem_wb).wait()
                gather(tt, cb, pb, sem_gb).start()
                gather(tt, cb, pb, sem_gb).wait()
                if p < 3:
                    write(tt, cb, pb, sem_wb).start()
                else:
                    @pl.when(tt % 2 == 0)
                    def _():
                        tail(tt, buf7a, sem_7a)

                    @pl.when(tt % 2 == 1)
                    def _():
                        tail(tt, buf7b, sem_7b)
                write(tt, ca, pa, sem_wa).wait()
                if p < 3:
                    gather(tt, ca + 2, pa, sem_ga).start()
                else:
                    @pl.when(tt < t - 1)
                    def _():
                        gather(tt + 1, 0, pa, sem_ga).start()
            return carry

        lax.fori_loop(0, t, body, 0)
        b7 = [buf7a, buf7b]
        s7 = [sem_7a, sem_7b]
        write7(t - 2, b7[(t - 2) % 2], s7[(t - 2) % 2]).wait()
        write7(t - 1, b7[(t - 1) % 2], s7[(t - 1) % 2]).wait()

    return sc_gather


def kernel(x, embedding):
    b, t = x.shape
    v, d = embedding.shape
    t8 = jnp.pad(embedding, ((0, 0), (0, D_PAD - d))).reshape(v * NCB, 128)
    xr = x.astype(jnp.int32).reshape(NUM_WORKERS, BPW, t).transpose(0, 2, 1)
    gidx = (xr[:, :, None, :] * NCB
            + jnp.arange(NCB, dtype=jnp.int32)[None, None, :, None])
    return _make_sc_gather(b, t, d)(t8, gidx)


# speedup vs baseline: 2.9756x; 1.0115x over previous
import jax
import jax.numpy as jnp
from jax import lax
from jax.experimental import pallas as pl
from jax.experimental.pallas import tpu as pltpu
from jax.experimental.pallas import tpu_sc as plsc

NUM_WORKERS = 32
BPW = 128
NCB = 8
D_PAD = NCB * 128


def _make_sc_gather(b, t, d):
    d_tail = d - (NCB - 1) * 128
    mesh = plsc.VectorSubcoreMesh(core_axis_name="c", subcore_axis_name="s")

    @pl.kernel(
        mesh=mesh,
        out_type=jax.ShapeDtypeStruct((b, t, d), jnp.float32),
        scratch_types=[
            pltpu.VMEM((t, NCB, BPW), jnp.int32),
            pltpu.VMEM((BPW, 128), jnp.float32),
            pltpu.VMEM((BPW, 128), jnp.float32),
            pltpu.VMEM((BPW, d_tail), jnp.float32),
            pltpu.VMEM((BPW, d_tail), jnp.float32),
            pltpu.SemaphoreType.DMA,
            pltpu.SemaphoreType.DMA,
            pltpu.SemaphoreType.DMA,
            pltpu.SemaphoreType.DMA,
            pltpu.SemaphoreType.DMA,
            pltpu.SemaphoreType.DMA,
        ],
    )
    def sc_gather(t8_hbm, idx_hbm, out_hbm, idx_v, pa, pb, buf7a, buf7b,
                  sem_ga, sem_gb, sem_wa, sem_wb, sem_7a, sem_7b):
        wid = lax.axis_index("s") * 2 + lax.axis_index("c")
        bb = wid * BPW
        pltpu.sync_copy(idx_hbm.at[wid], idx_v)

        def gather(tt, c, buf, sem):
            return pltpu.make_async_copy(t8_hbm.at[idx_v.at[tt, c]], buf, sem)

        def write(tt, c, buf, sem):
            return pltpu.make_async_copy(
                buf, out_hbm.at[pl.ds(bb, BPW), tt, pl.ds(c * 128, 128)], sem)

        def write7(tt, bf, sem):
            return pltpu.make_async_copy(
                bf, out_hbm.at[pl.ds(bb, BPW), tt, pl.ds((NCB - 1) * 128,
                                                         d_tail)], sem)

        def tail(tt, bf, sem):
            @pl.when(tt >= 2)
            def _():
                write7(tt - 2, bf, sem).wait()

            def row_copy(r, carry):
                for kk in range(d_tail // 16):
                    bf[r, pl.ds(kk * 16, 16)] = pb[r, pl.ds(kk * 16, 16)]
                bf[r, pl.ds(d_tail - 16, 16)] = pb[r, pl.ds(d_tail - 16, 16)]
                return carry
            lax.fori_loop(0, BPW, row_copy, 0)
            write7(tt, bf, sem).start()

        gather(0, 0, pa, sem_ga).start()
        bufs = [(pa, sem_ga, sem_wa), (pb, sem_gb, sem_wb)]

        def body(tt, carry):
            for c in range(NCB):
                cur, sg, sw = bufs[c % 2]
                oth, sgo, swo = bufs[1 - c % 2]
                gather(tt, c, cur, sg).wait()
                if c < NCB - 1:
                    write(tt, c, cur, sw).start()
                if c >= 1:
                    write(tt, c - 1, oth, swo).wait()
                if c < NCB - 1:
                    gather(tt, c + 1, oth, sgo).start()
                else:
                    @pl.when(tt < t - 1)
                    def _():
                        gather(tt + 1, 0, pa, sem_ga).start()

                    @pl.when(tt % 2 == 0)
                    def _():
                        tail(tt, buf7a, sem_7a)

                    @pl.when(tt % 2 == 1)
                    def _():
                        tail(tt, buf7b, sem_7b)
            return carry

        lax.fori_loop(0, t, body, 0)
        b7 = [buf7a, buf7b]
        s7 = [sem_7a, sem_7b]
        write7(t - 2, b7[(t - 2) % 2], s7[(t - 2) % 2]).wait()
        write7(t - 1, b7[(t - 1) % 2], s7[(t - 1) % 2]).wait()

    return sc_gather


def kernel(x, embedding):
    b, t = x.shape
    v, d = embedding.shape
    t8 = jnp.pad(embedding, ((0, 0), (0, D_PAD - d))).reshape(v * NCB, 128)
    xr = x.astype(jnp.int32).reshape(NUM_WORKERS, BPW, t).transpose(0, 2, 1)
    gidx = (xr[:, :, None, :] * NCB
            + jnp.arange(NCB, dtype=jnp.int32)[None, None, :, None])
    return _make_sc_gather(b, t, d)(t8, gidx)
